# baseline (speedup 1.0000x reference)
import functools

import jax
import jax.numpy as jnp
import numpy as np
from jax import lax
from jax.experimental import pallas as pl
from jax.experimental.pallas import tpu as pltpu
from jax.experimental.pallas import tpu_sc as plsc

_VOCAB = 30522
_MASK_ID = 103
_RATE = 0.15
_MAX_SEL = 614
_B = 16
_S = 4096
_L = 16
_PAD = 640


def _tf2x32(k1, k2, x0, x1):
    rot = [np.uint32(r) for r in (13, 15, 26, 6, 17, 29, 16, 24)]
    ks = [np.uint32(k1), np.uint32(k2),
          np.uint32(k1 ^ k2 ^ np.uint32(0x1BD11BDA))]
    x = [x0.astype(np.uint32) + ks[0], x1.astype(np.uint32) + ks[1]]

    def rounds(rs):
        for r in rs:
            x[0] = x[0] + x[1]
            x[1] = (x[1] << r) | (x[1] >> (np.uint32(32) - r))
            x[1] = x[0] ^ x[1]

    with np.errstate(over="ignore"):
        rounds(rot[:4]); x[0] += ks[1]; x[1] += ks[2] + np.uint32(1)
        rounds(rot[4:]); x[0] += ks[2]; x[1] += ks[0] + np.uint32(2)
        rounds(rot[:4]); x[0] += ks[0]; x[1] += ks[1] + np.uint32(3)
        rounds(rot[4:]); x[0] += ks[1]; x[1] += ks[2] + np.uint32(4)
        rounds(rot[:4]); x[0] += ks[2]; x[1] += ks[0] + np.uint32(5)
    return x[0], x[1]


def _iota2x32(shape):
    i = np.arange(int(np.prod(shape)), dtype=np.uint64)
    return ((i >> np.uint64(32)).astype(np.uint32).reshape(shape),
            (i & np.uint64(0xFFFFFFFF)).astype(np.uint32).reshape(shape))


def _split_key(key, num):
    b1, b2 = _tf2x32(key[0], key[1], *_iota2x32((num,)))
    return np.stack([b1, b2], axis=1)


def _random_bits(key, shape):
    b1, b2 = _tf2x32(key[0], key[1], *_iota2x32(shape))
    return b1 ^ b2


def _uniform01(key, shape):
    bits = _random_bits(key, shape)
    fb = (bits >> np.uint32(9)) | np.uint32(0x3F800000)
    return fb.view(np.float32) - np.float32(1.0)


def _randint(key, shape, span):
    ks = _split_key(key, 2)
    hi, lo = _random_bits(ks[0], shape), _random_bits(ks[1], shape)
    span = np.uint32(span)
    with np.errstate(over="ignore"):
        mult = np.uint32(65536) % span
        mult = (mult * mult) % span
        off = ((hi % span) * mult + lo % span) % span
    return off.astype(np.int32)


@functools.lru_cache(maxsize=None)
def _consts():
    k_sel, k_choice, k_rand = _split_key(np.array([0, 42], np.uint32), 3)
    scores = _uniform01(k_sel, (_B, _S))
    perm = np.argsort(-scores, axis=1, kind="stable").astype(np.int32)
    r = _uniform01(k_choice, (_B, _MAX_SEL))
    rand_tok = _randint(k_rand, (_B, _MAX_SEL), _VOCAB)
    nvb = np.where(r < np.float32(0.8), _MASK_ID,
                   np.where(r < np.float32(0.9), rand_tok, -1)).astype(np.int32)
    nvb = np.pad(nvb, ((0, 0), (0, _PAD - _MAX_SEL)))
    skey = scores.view(np.int32)
    return perm, nvb, skey


_NCH = _S // _L
_NO = _NCH // _L


def _masker_body(inp_hbm, perm_hbm, nvb_hbm, skey_hbm,
                 out0_hbm, out1_hbm, out2_hbm,
                 inp_v, perm_v, selp_v, skey_v, nvb_v, outrow_v, pos_v, ids_v,
                 cnt_v, tmp_v):
    c = lax.axis_index("c")
    s = lax.axis_index("s")

    @pl.when(c == 0)
    def _():
        b = s
        pltpu.sync_copy(inp_hbm.at[b], inp_v)
        pltpu.sync_copy(perm_hbm.at[b], perm_v)
        pltpu.sync_copy(nvb_hbm.at[b], nvb_v)
        pltpu.sync_copy(skey_hbm.at[b], skey_v)

        iota = lax.iota(jnp.int32, _L)
        lane0 = iota == 0


        @plsc.parallel_loop(0, _NCH, unroll=8)
        def _pa1(j):
            idx = perm_v[pl.ds(j * _L, _L)]
            tp = plsc.load_gather(inp_v, [idx])
            sb = (tp != 0) & (tp != 101) & (tp != 102)
            selp_v[pl.ds(j * _L, _L)] = sb.astype(jnp.int32)
            cpc = plsc.all_reduce_population_count(sb)
            plsc.store_scatter(cnt_v, [jnp.broadcast_to(j, (_L,))], cpc,
                               mask=lane0)

        def pa2(i, cnt):
            v = cnt_v[pl.ds(i * _L, _L)]
            cs = plsc.cumsum(v)
            cnt_v[pl.ds(i * _L, _L)] = cs - v + cnt
            return cnt + cs[_L - 1]

        ns = lax.fori_loop(0, _NO, pa2, jnp.int32(0))
        k1 = (jnp.float32(_RATE) * ns.astype(jnp.float32)).astype(jnp.int32)
        kk = jnp.minimum(jnp.minimum(jnp.int32(_MAX_SEL), k1), ns)

        tmp_v[pl.ds(0, _L)] = jnp.where(
            lane0, jnp.int32(0x7FFFFFFF),
            jnp.where(iota == 1, jnp.int32(-1), jnp.int32(0)))

        @pl.when(kk > 0)
        def _():
            def srch(i, acc):
                v = cnt_v[pl.ds(i * _L, _L)]
                cm = v <= (kk - 1)
                return acc + plsc.all_reduce_population_count(cm)[0]

            jstar = lax.fori_loop(0, _NO, srch, jnp.int32(0)) - 1
            base = cnt_v[pl.ds(jstar, _L)][0]
            sf = selp_v[pl.ds(jstar * _L, _L)]
            cs = plsc.cumsum(sf)
            hit = (sf == 1) & ((base + cs) == kk)
            idx = perm_v[pl.ds(jstar * _L, _L)]
            keyv = plsc.load_gather(skey_v, [idx])
            plsc.store_scatter(tmp_v, [jnp.broadcast_to(0, (_L,))], keyv,
                               mask=hit)
            plsc.store_scatter(tmp_v, [jnp.broadcast_to(1, (_L,))], idx,
                               mask=hit)

        tv = tmp_v[pl.ds(0, _L)]
        pkey = tv[0]
        ppos = tv[1]

        @plsc.parallel_loop(0, _PAD // _L, unroll=8)
        def _pz(i):
            zero = jnp.zeros((_L,), jnp.int32)
            pos_v[pl.ds(i * _L, _L)] = zero
            ids_v[pl.ds(i * _L, _L)] = zero

        @plsc.parallel_loop(0, _NCH, unroll=8)
        def _pb1(i):
            off = i * _L
            tt = inp_v[pl.ds(off, _L)]
            sb = (tt != 0) & (tt != 101) & (tt != 102)
            kb = skey_v[pl.ds(off, _L)]
            posv = iota + off
            m = sb & ((kb > pkey) | ((kb == pkey) & (posv <= ppos)))
            selp_v[pl.ds(off, _L)] = m.astype(jnp.int32)
            cpc = plsc.all_reduce_population_count(m)
            plsc.store_scatter(cnt_v, [jnp.broadcast_to(i, (_L,))], cpc,
                               mask=lane0)

        def pb2(i, cnt):
            v = cnt_v[pl.ds(i * _L, _L)]
            cs = plsc.cumsum(v)
            cnt_v[pl.ds(i * _L, _L)] = cs - v + cnt
            return cnt + cs[_L - 1]

        lax.fori_loop(0, _NO, pb2, jnp.int32(0))

        @plsc.parallel_loop(0, _NO, unroll=1)
        def _pb3(o):
            cvec = cnt_v[pl.ds(o * _L, _L)]
            for k in range(_L):
                i = o * _L + k
                off = i * _L
                sd = selp_v[pl.ds(off, _L)]
                m = sd == 1
                cs = plsc.cumsum(sd)
                slot = cs - sd + cvec[k]
                tt = inp_v[pl.ds(off, _L)]
                posv = iota + off
                plsc.store_scatter(pos_v, [slot], posv, mask=m)
                plsc.store_scatter(ids_v, [slot], tt, mask=m)
                nv = plsc.load_gather(nvb_v, [slot], mask=m)
                outc = jnp.where(m & (nv != -1), nv, tt)
                outrow_v[pl.ds(off, _L)] = outc

        pltpu.sync_copy(outrow_v, out0_hbm.at[b])
        pltpu.sync_copy(pos_v, out1_hbm.at[b])
        pltpu.sync_copy(ids_v, out2_hbm.at[b])


@jax.jit
def _masker(inputs, perm, nvb, skey):
    mesh = plsc.VectorSubcoreMesh(core_axis_name="c", subcore_axis_name="s", num_cores=1)
    fn = pl.kernel(
        _masker_body,
        mesh=mesh,
        compiler_params=pltpu.CompilerParams(needs_layout_passes=False),
        out_type=[
            jax.ShapeDtypeStruct((_B, _S), jnp.int32),
            jax.ShapeDtypeStruct((_B, _PAD), jnp.int32),
            jax.ShapeDtypeStruct((_B, _PAD), jnp.int32),
        ],
        scratch_types=[
            pltpu.VMEM((_S,), jnp.int32),
            pltpu.VMEM((_S,), jnp.int32),
            pltpu.VMEM((_S,), jnp.int32),
            pltpu.VMEM((_S,), jnp.int32),
            pltpu.VMEM((_PAD,), jnp.int32),
            pltpu.VMEM((_S,), jnp.int32),
            pltpu.VMEM((_PAD,), jnp.int32),
            pltpu.VMEM((_PAD,), jnp.int32),
            pltpu.VMEM((_NCH + _L,), jnp.int32),
            pltpu.VMEM((_L,), jnp.int32),
        ],
    )
    return fn(inputs, perm, nvb, skey)


def kernel(inputs):
    perm, nvb, skey = _consts()
    out0, out1p, out2p = _masker(inputs, jnp.asarray(perm), jnp.asarray(nvb),
                                 jnp.asarray(skey))
    return out0, out1p[:, :_MAX_SEL], out2p[:, :_MAX_SEL]


# speedup vs baseline: 1.0931x; 1.0931x over previous
import functools

import jax
import jax.numpy as jnp
import numpy as np
from jax import lax
from jax.experimental import pallas as pl
from jax.experimental.pallas import tpu as pltpu
from jax.experimental.pallas import tpu_sc as plsc

_VOCAB = 30522
_MASK_ID = 103
_RATE = 0.15
_MAX_SEL = 614
_B = 16
_S = 4096
_L = 16
_PAD = 640


def _tf2x32(k1, k2, x0, x1):
    rot = [np.uint32(r) for r in (13, 15, 26, 6, 17, 29, 16, 24)]
    ks = [np.uint32(k1), np.uint32(k2),
          np.uint32(k1 ^ k2 ^ np.uint32(0x1BD11BDA))]
    x = [x0.astype(np.uint32) + ks[0], x1.astype(np.uint32) + ks[1]]

    def rounds(rs):
        for r in rs:
            x[0] = x[0] + x[1]
            x[1] = (x[1] << r) | (x[1] >> (np.uint32(32) - r))
            x[1] = x[0] ^ x[1]

    with np.errstate(over="ignore"):
        rounds(rot[:4]); x[0] += ks[1]; x[1] += ks[2] + np.uint32(1)
        rounds(rot[4:]); x[0] += ks[2]; x[1] += ks[0] + np.uint32(2)
        rounds(rot[:4]); x[0] += ks[0]; x[1] += ks[1] + np.uint32(3)
        rounds(rot[4:]); x[0] += ks[1]; x[1] += ks[2] + np.uint32(4)
        rounds(rot[:4]); x[0] += ks[2]; x[1] += ks[0] + np.uint32(5)
    return x[0], x[1]


def _iota2x32(shape):
    i = np.arange(int(np.prod(shape)), dtype=np.uint64)
    return ((i >> np.uint64(32)).astype(np.uint32).reshape(shape),
            (i & np.uint64(0xFFFFFFFF)).astype(np.uint32).reshape(shape))


def _split_key(key, num):
    b1, b2 = _tf2x32(key[0], key[1], *_iota2x32((num,)))
    return np.stack([b1, b2], axis=1)


def _random_bits(key, shape):
    b1, b2 = _tf2x32(key[0], key[1], *_iota2x32(shape))
    return b1 ^ b2


def _uniform01(key, shape):
    bits = _random_bits(key, shape)
    fb = (bits >> np.uint32(9)) | np.uint32(0x3F800000)
    return fb.view(np.float32) - np.float32(1.0)


def _randint(key, shape, span):
    ks = _split_key(key, 2)
    hi, lo = _random_bits(ks[0], shape), _random_bits(ks[1], shape)
    span = np.uint32(span)
    with np.errstate(over="ignore"):
        mult = np.uint32(65536) % span
        mult = (mult * mult) % span
        off = ((hi % span) * mult + lo % span) % span
    return off.astype(np.int32)


@functools.lru_cache(maxsize=None)
def _consts():
    k_sel, k_choice, k_rand = _split_key(np.array([0, 42], np.uint32), 3)
    scores = _uniform01(k_sel, (_B, _S))
    perm = np.argsort(-scores, axis=1, kind="stable").astype(np.int32)
    r = _uniform01(k_choice, (_B, _MAX_SEL))
    rand_tok = _randint(k_rand, (_B, _MAX_SEL), _VOCAB)
    nvb = np.where(r < np.float32(0.8), _MASK_ID,
                   np.where(r < np.float32(0.9), rand_tok, -1)).astype(np.int32)
    nvb = np.pad(nvb, ((0, 0), (0, _PAD - _MAX_SEL)))
    skey = scores.view(np.int32)
    return perm, nvb, skey


_NCH = _S // _L
_NO = _NCH // _L


def _masker_body(inp_hbm, perm_hbm, nvb_hbm, skey_hbm,
                 out0_hbm, out1_hbm, out2_hbm,
                 inp_v, perm_v, selp_v, skey_v, nvb_v, outrow_v, pos_v, ids_v,
                 cnt_v, tmp_v):
    c = lax.axis_index("c")
    s = lax.axis_index("s")

    @pl.when(c == 0)
    def _():
        b = s
        pltpu.sync_copy(inp_hbm.at[b], inp_v)
        pltpu.sync_copy(perm_hbm.at[b], perm_v)
        pltpu.sync_copy(nvb_hbm.at[b], nvb_v)
        pltpu.sync_copy(skey_hbm.at[b], skey_v)

        iota = lax.iota(jnp.int32, _L)
        lane0 = iota == 0


        @plsc.parallel_loop(0, _NCH, unroll=8)
        def _pa1(j):
            idx = perm_v[pl.ds(j * _L, _L)]
            tp = plsc.load_gather(inp_v, [idx])
            sb = (tp != 0) & (tp != 101) & (tp != 102)
            selp_v[pl.ds(j * _L, _L)] = sb.astype(jnp.int32)
            cpc = plsc.all_reduce_population_count(sb)
            plsc.store_scatter(cnt_v, [jnp.broadcast_to(j, (_L,))], cpc,
                               mask=lane0)

        def pa2(i, cnt):
            v = cnt_v[pl.ds(i * _L, _L)]
            cs = plsc.cumsum(v)
            cnt_v[pl.ds(i * _L, _L)] = cs - v + cnt
            return cnt + cs[_L - 1]

        ns = lax.fori_loop(0, _NO, pa2, jnp.int32(0))
        k1 = (jnp.float32(_RATE) * ns.astype(jnp.float32)).astype(jnp.int32)
        kk = jnp.minimum(jnp.minimum(jnp.int32(_MAX_SEL), k1), ns)

        tmp_v[pl.ds(0, _L)] = jnp.where(
            lane0, jnp.int32(0x7FFFFFFF),
            jnp.where(iota == 1, jnp.int32(-1), jnp.int32(0)))

        @pl.when(kk > 0)
        def _():
            def srch(i, acc):
                v = cnt_v[pl.ds(i * _L, _L)]
                cm = v <= (kk - 1)
                return acc + plsc.all_reduce_population_count(cm)[0]

            jstar = lax.fori_loop(0, _NO, srch, jnp.int32(0)) - 1
            base = cnt_v[pl.ds(jstar, _L)][0]
            sf = selp_v[pl.ds(jstar * _L, _L)]
            cs = plsc.cumsum(sf)
            hit = (sf == 1) & ((base + cs) == kk)
            idx = perm_v[pl.ds(jstar * _L, _L)]
            keyv = plsc.load_gather(skey_v, [idx])
            plsc.store_scatter(tmp_v, [jnp.broadcast_to(0, (_L,))], keyv,
                               mask=hit)
            plsc.store_scatter(tmp_v, [jnp.broadcast_to(1, (_L,))], idx,
                               mask=hit)

        tv = tmp_v[pl.ds(0, _L)]
        pkey = tv[0]
        ppos = tv[1]

        @plsc.parallel_loop(0, _PAD // _L, unroll=8)
        def _pz(i):
            zero = jnp.zeros((_L,), jnp.int32)
            pos_v[pl.ds(i * _L, _L)] = zero
            ids_v[pl.ds(i * _L, _L)] = zero

        @plsc.parallel_loop(0, _NCH, unroll=8)
        def _pb1(i):
            off = i * _L
            tt = inp_v[pl.ds(off, _L)]
            sb = (tt != 0) & (tt != 101) & (tt != 102)
            kb = skey_v[pl.ds(off, _L)]
            posv = iota + off
            m = sb & ((kb > pkey) | ((kb == pkey) & (posv <= ppos)))
            selp_v[pl.ds(off, _L)] = m.astype(jnp.int32)
            cpc = plsc.all_reduce_population_count(m)
            plsc.store_scatter(cnt_v, [jnp.broadcast_to(i, (_L,))], cpc,
                               mask=lane0)

        def pb2(i, cnt):
            v = cnt_v[pl.ds(i * _L, _L)]
            cs = plsc.cumsum(v)
            cnt_v[pl.ds(i * _L, _L)] = cs - v + cnt
            return cnt + cs[_L - 1]

        lax.fori_loop(0, _NO, pb2, jnp.int32(0))

        @plsc.parallel_loop(0, _NCH, unroll=4)
        def _pb3(i):
            off = i * _L
            sd = selp_v[pl.ds(off, _L)]
            m = sd == 1
            cs = plsc.cumsum(sd)
            slot = cs - sd + cnt_v[pl.ds(i, _L)][0]
            tt = inp_v[pl.ds(off, _L)]
            posv = iota + off
            plsc.store_scatter(pos_v, [slot], posv, mask=m)
            plsc.store_scatter(ids_v, [slot], tt, mask=m)
            nv = plsc.load_gather(nvb_v, [slot], mask=m)
            outc = jnp.where(m & (nv != -1), nv, tt)
            outrow_v[pl.ds(off, _L)] = outc

        pltpu.sync_copy(outrow_v, out0_hbm.at[b])
        pltpu.sync_copy(pos_v, out1_hbm.at[b])
        pltpu.sync_copy(ids_v, out2_hbm.at[b])


@jax.jit
def _masker(inputs, perm, nvb, skey):
    mesh = plsc.VectorSubcoreMesh(core_axis_name="c", subcore_axis_name="s", num_cores=1)
    fn = pl.kernel(
        _masker_body,
        mesh=mesh,
        compiler_params=pltpu.CompilerParams(needs_layout_passes=False),
        out_type=[
            jax.ShapeDtypeStruct((_B, _S), jnp.int32),
            jax.ShapeDtypeStruct((_B, _PAD), jnp.int32),
            jax.ShapeDtypeStruct((_B, _PAD), jnp.int32),
        ],
        scratch_types=[
            pltpu.VMEM((_S,), jnp.int32),
            pltpu.VMEM((_S,), jnp.int32),
            pltpu.VMEM((_S,), jnp.int32),
            pltpu.VMEM((_S,), jnp.int32),
            pltpu.VMEM((_PAD,), jnp.int32),
            pltpu.VMEM((_S,), jnp.int32),
            pltpu.VMEM((_PAD,), jnp.int32),
            pltpu.VMEM((_PAD,), jnp.int32),
            pltpu.VMEM((_NCH + _L,), jnp.int32),
            pltpu.VMEM((_L,), jnp.int32),
        ],
    )
    return fn(inputs, perm, nvb, skey)


def kernel(inputs):
    perm, nvb, skey = _consts()
    out0, out1p, out2p = _masker(inputs, jnp.asarray(perm), jnp.asarray(nvb),
                                 jnp.asarray(skey))
    return out0, out1p[:, :_MAX_SEL], out2p[:, :_MAX_SEL]
